# baseline (speedup 1.0000x reference)
import functools

import jax
import jax.numpy as jnp
from jax import lax
from jax.experimental import pallas as pl
from jax.experimental.pallas import tpu as pltpu

B = 1024
D = 32
H = 64
N_ITEMS = 100000
N_TOP = 2000
N_MID = 10000
N_USERS = 256
K = 20

BB = 256
CHUNK = 2048
QROWS = CHUNK // 128
N_CHUNKS = (N_ITEMS + CHUNK - 1) // CHUNK
PAD_LANES = 128
IDX_SENT = 2**31 - 1
MAP_SENT = 1 << 29
NEG_INF = float("-inf")


def _topk_extract(V, I, k):
    outs_v, outs_i = [], []
    for _ in range(k):
        m = jnp.max(V, axis=1, keepdims=True)
        tie = V == m
        ci = jnp.where(tie, I, IDX_SENT)
        si = jnp.min(ci, axis=1, keepdims=True)
        outs_v.append(m)
        outs_i.append(si)
        V = jnp.where(tie & (I == si), NEG_INF, V)
    return jnp.concatenate(outs_v, axis=1), jnp.concatenate(outs_i, axis=1)


def _select_group_lanes(gm, gbest):
    bb = gm.shape[0]
    _, sel_best = _topk_extract(gm, gbest, K)
    lane = lax.broadcasted_iota(jnp.int32, (bb, PAD_LANES), 1)
    sel_lane_list = []
    for t in range(K):
        hit = gbest == sel_best[:, t:t + 1]
        sel_lane_list.append(
            jnp.min(jnp.where(hit, lane, IDX_SENT), axis=1, keepdims=True))
    return jnp.concatenate(sel_lane_list, axis=1)


def _merge_chunk(scores, i3_rows, prev_v, prev_i):
    bb = scores.shape[0]
    s3 = scores.reshape(bb, QROWS, PAD_LANES)
    gm = jnp.max(s3, axis=1)
    tie3 = s3 == gm[:, None, :]
    i3 = jnp.stack(i3_rows, axis=1)
    gbest = jnp.min(jnp.where(tie3, i3, IDX_SENT), axis=1)
    sel_lane = _select_group_lanes(gm, gbest)
    cand_v = [prev_v]
    cand_i = [prev_i]
    for q in range(QROWS):
        cand_v.append(jnp.take_along_axis(s3[:, q, :], sel_lane, axis=1))
        cand_i.append(jnp.take_along_axis(i3_rows[q], sel_lane, axis=1))
    V = jnp.concatenate(cand_v, axis=1)
    I = jnp.concatenate(cand_i, axis=1)
    return _topk_extract(V, I, K)


def _pad_state(vals, idx):
    bb = vals.shape[0]
    return (jnp.concatenate(
                [vals, jnp.full((bb, PAD_LANES - K), NEG_INF, jnp.float32)],
                axis=1),
            jnp.concatenate(
                [idx, jnp.full((bb, PAD_LANES - K), IDX_SENT, jnp.int32)],
                axis=1))


def _subset_kernel(x_ref, wp_ref, wd_ref, map_ref, zc_ref, out_ref,
                   h_ref, vals_ref):
    b, j = pl.program_id(0), pl.program_id(1)

    @pl.when(j == 0)
    def _():
        h_ref[...] = jnp.tanh(
            lax.dot_general(x_ref[...], wp_ref[...], (((1,), (0,)), ((), ())),
                            preferred_element_type=jnp.float32))

    preds = lax.dot_general(h_ref[...], wd_ref[...], (((1,), (0,)), ((), ())),
                            preferred_element_type=jnp.float32)
    bb = preds.shape[0]
    i3_rows = [
        jnp.broadcast_to(map_ref[:, q * 128:(q + 1) * 128], (bb, 128))
        for q in range(QROWS)
    ]
    first = j == 0
    prev_v = jnp.where(first, 0.0, vals_ref[...])
    prev_i = jnp.where(first, jnp.broadcast_to(zc_ref[...], (bb, PAD_LANES)),
                       out_ref[...])
    vals, idx = _merge_chunk(preds, i3_rows, prev_v, prev_i)
    vals_ref[...], out_ref[...] = _pad_state(vals, idx)


def _subset_topk(X, W_p, W_d, idx_map, zc, n_sub_pad):
    n_chunks = n_sub_pad // CHUNK
    call = pl.pallas_call(
        _subset_kernel,
        grid=(B // BB, n_chunks),
        in_specs=[
            pl.BlockSpec((BB, D), lambda b, j: (b, 0)),
            pl.BlockSpec((D, H), lambda b, j: (0, 0)),
            pl.BlockSpec((H, CHUNK), lambda b, j: (0, j)),
            pl.BlockSpec((1, CHUNK), lambda b, j: (0, j)),
            pl.BlockSpec((1, PAD_LANES), lambda b, j: (0, 0)),
        ],
        out_specs=pl.BlockSpec((BB, PAD_LANES), lambda b, j: (b, 0)),
        out_shape=jax.ShapeDtypeStruct((B, PAD_LANES), jnp.int32),
        scratch_shapes=[
            pltpu.VMEM((BB, H), jnp.float32),
            pltpu.VMEM((BB, PAD_LANES), jnp.float32),
        ],
    )
    return call(X, W_p, W_d, idx_map, zc)[:, :K]


def _knn_kernel(x_ref, p_ref, r_ref, out_ref, w_ref, vals_ref):
    b, j = pl.program_id(0), pl.program_id(1)

    @pl.when(j == 0)
    def _():
        x = x_ref[...]
        xn = x / (jnp.sqrt(jnp.sum(x * x, axis=1, keepdims=True)) + 1e-8)
        p = p_ref[...]
        pn = p / (jnp.sqrt(jnp.sum(p * p, axis=1, keepdims=True)) + 1e-8)
        sims = lax.dot_general(xn, pn, (((1,), (1,)), ((), ())),
                               preferred_element_type=jnp.float32)
        w_ref[...] = jnp.maximum(sims, 0.0)

    scores = lax.dot_general(w_ref[...], r_ref[...], (((1,), (0,)), ((), ())),
                             preferred_element_type=jnp.float32)
    bb = scores.shape[0]
    gcol = j * CHUNK + lax.broadcasted_iota(jnp.int32, (bb, CHUNK), 1)
    scores = jnp.where(gcol < N_ITEMS, scores, NEG_INF)
    i3_rows = [gcol[:, q * 128:(q + 1) * 128] for q in range(QROWS)]

    first = j == 0
    prev_v = jnp.where(first, NEG_INF, vals_ref[...])
    prev_i = jnp.where(first, IDX_SENT, out_ref[...])
    vals, idx = _merge_chunk(scores, i3_rows, prev_v, prev_i)
    vals_ref[...], out_ref[...] = _pad_state(vals, idx)


def _knn_topk(X, user_ratings, user_personalities):
    call = pl.pallas_call(
        _knn_kernel,
        grid=(B // BB, N_CHUNKS),
        in_specs=[
            pl.BlockSpec((BB, D), lambda b, j: (b, 0)),
            pl.BlockSpec((N_USERS, D), lambda b, j: (0, 0)),
            pl.BlockSpec((N_USERS, CHUNK), lambda b, j: (0, j)),
        ],
        out_specs=pl.BlockSpec((BB, PAD_LANES), lambda b, j: (b, 0)),
        out_shape=jax.ShapeDtypeStruct((B, PAD_LANES), jnp.int32),
        scratch_shapes=[
            pltpu.VMEM((BB, N_USERS), jnp.float32),
            pltpu.VMEM((BB, PAD_LANES), jnp.float32),
        ],
    )
    return call(X, user_personalities, user_ratings)[:, :K]


def _smallest_unmapped(idx_map):
    present = jnp.zeros((N_ITEMS,), jnp.int32).at[idx_map].set(1)
    score = jnp.arange(N_ITEMS, dtype=jnp.int32) + present * (2 * N_ITEMS)
    neg_top, _ = lax.top_k(-score, PAD_LANES)
    return (-neg_top).reshape(1, PAD_LANES)


def kernel(X, W_sp, W_sd, W_mp, W_md, user_ratings, user_personalities,
           top_map, mid_map):
    top_map = top_map.astype(jnp.int32)
    mid_map = mid_map.astype(jnp.int32)

    n_top_pad = 2048
    n_mid_pad = 10240
    W_sd_p = jnp.pad(W_sd, ((0, 0), (0, n_top_pad - N_TOP)))
    W_md_p = jnp.pad(W_md, ((0, 0), (0, n_mid_pad - N_MID)))
    top_map_p = jnp.pad(top_map, (0, n_top_pad - N_TOP),
                        constant_values=MAP_SENT).reshape(1, n_top_pad)
    mid_map_p = jnp.pad(mid_map, (0, n_mid_pad - N_MID),
                        constant_values=MAP_SENT).reshape(1, n_mid_pad)
    zc_top = _smallest_unmapped(top_map)
    zc_mid = _smallest_unmapped(mid_map)

    top_idx = _subset_topk(X, W_sp, W_sd_p, top_map_p, zc_top, n_top_pad)
    mid_idx = _subset_topk(X, W_mp, W_md_p, mid_map_p, zc_mid, n_mid_pad)
    k_idx = _knn_topk(X, user_ratings, user_personalities)

    return jnp.concatenate(
        [top_idx[:, None, :], mid_idx[:, None, :], k_idx[:, None, :]], axis=1)


# speedup vs baseline: 2.3619x; 2.3619x over previous
import functools

import jax
import jax.numpy as jnp
from jax import lax
from jax.experimental import pallas as pl
from jax.experimental.pallas import tpu as pltpu

B = 1024
D = 32
H = 64
N_ITEMS = 100000
N_TOP = 2000
N_MID = 10000
N_USERS = 256
K = 20

BB = 256
CHUNK = 2048
QROWS = CHUNK // 128
N_CHUNKS = (N_ITEMS + CHUNK - 1) // CHUNK
PAD_LANES = 128
IDX_SENT = 2**31 - 1
MAP_SENT = 1 << 29
NEG_INF = float("-inf")


def _topk_extract(V, I, k):
    outs_v, outs_i = [], []
    for _ in range(k):
        m = jnp.max(V, axis=1, keepdims=True)
        tie = V == m
        ci = jnp.where(tie, I, IDX_SENT)
        si = jnp.min(ci, axis=1, keepdims=True)
        outs_v.append(m)
        outs_i.append(si)
        V = jnp.where(tie & (I == si), NEG_INF, V)
    return jnp.concatenate(outs_v, axis=1), jnp.concatenate(outs_i, axis=1)


def _ce(H, HI, i, j):
    xv, xi, yv, yi = H[i], HI[i], H[j], HI[j]
    takex = (xv > yv) | ((xv == yv) & (xi < yi))
    H[i] = jnp.where(takex, xv, yv)
    HI[i] = jnp.where(takex, xi, yi)
    H[j] = jnp.where(takex, yv, xv)
    HI[j] = jnp.where(takex, yi, xi)


def _merge_chunk(scores, idx_cols, prev_v, prev_i):
    bb, n = scores.shape
    n4 = n // 4
    H = [scores[:, t * n4:(t + 1) * n4] for t in range(4)]
    HI = list(idx_cols)
    for (i, j) in ((0, 1), (2, 3), (0, 2), (1, 3), (1, 2)):
        _ce(H, HI, i, j)
    Bv, Bi = H[0], HI[0]
    Av, Ai = prev_v, prev_i
    outs_v, outs_i = [], []
    for _ in range(K):
        mB = jnp.max(Bv, axis=1, keepdims=True)
        mA = jnp.max(Av, axis=1, keepdims=True)
        m = jnp.maximum(mA, mB)
        tieB = Bv == m
        tieA = Av == m
        siB = jnp.min(jnp.where(tieB, Bi, IDX_SENT), axis=1, keepdims=True)
        siA = jnp.min(jnp.where(tieA, Ai, IDX_SENT), axis=1, keepdims=True)
        si = jnp.minimum(siA, siB)
        outs_v.append(m)
        outs_i.append(si)
        killA = tieA & (Ai == si)
        Av = jnp.where(killA, NEG_INF, Av)
        killB = tieB & (Bi == si)
        nv = jnp.where(Bi == HI[0], H[1],
                       jnp.where(Bi == HI[1], H[2],
                                 jnp.where(Bi == HI[2], H[3], NEG_INF)))
        ni = jnp.where(Bi == HI[0], HI[1],
                       jnp.where(Bi == HI[1], HI[2],
                                 jnp.where(Bi == HI[2], HI[3], IDX_SENT)))
        Bv = jnp.where(killB, nv, Bv)
        Bi = jnp.where(killB, ni, Bi)
    return jnp.concatenate(outs_v, axis=1), jnp.concatenate(outs_i, axis=1)


def _pad_state(vals, idx):
    bb = vals.shape[0]
    return (jnp.concatenate(
                [vals, jnp.full((bb, PAD_LANES - K), NEG_INF, jnp.float32)],
                axis=1),
            jnp.concatenate(
                [idx, jnp.full((bb, PAD_LANES - K), IDX_SENT, jnp.int32)],
                axis=1))


def _subset_kernel(x_ref, wp_ref, wd_ref, map_ref, zc_ref, out_ref,
                   h_ref, vals_ref):
    b, j = pl.program_id(0), pl.program_id(1)

    @pl.when(j == 0)
    def _():
        h_ref[...] = jnp.tanh(
            lax.dot_general(x_ref[...], wp_ref[...], (((1,), (0,)), ((), ())),
                            preferred_element_type=jnp.float32))

    preds = lax.dot_general(h_ref[...], wd_ref[...], (((1,), (0,)), ((), ())),
                            preferred_element_type=jnp.float32)
    bb, n = preds.shape
    n4 = n // 4
    idx_cols = [
        jnp.broadcast_to(map_ref[:, t * n4:(t + 1) * n4], (bb, n4))
        for t in range(4)
    ]
    first = j == 0
    prev_v = jnp.where(first, 0.0, vals_ref[...])
    prev_i = jnp.where(first, jnp.broadcast_to(zc_ref[...], (bb, PAD_LANES)),
                       out_ref[...])
    vals, idx = _merge_chunk(preds, idx_cols, prev_v, prev_i)
    vals_ref[...], out_ref[...] = _pad_state(vals, idx)


def _subset_topk(X, W_p, W_d, idx_map, zc, n_sub_pad):
    n_chunks = n_sub_pad // CHUNK
    call = pl.pallas_call(
        _subset_kernel,
        grid=(B // BB, n_chunks),
        in_specs=[
            pl.BlockSpec((BB, D), lambda b, j: (b, 0)),
            pl.BlockSpec((D, H), lambda b, j: (0, 0)),
            pl.BlockSpec((H, CHUNK), lambda b, j: (0, j)),
            pl.BlockSpec((1, CHUNK), lambda b, j: (0, j)),
            pl.BlockSpec((1, PAD_LANES), lambda b, j: (0, 0)),
        ],
        out_specs=pl.BlockSpec((BB, PAD_LANES), lambda b, j: (b, 0)),
        out_shape=jax.ShapeDtypeStruct((B, PAD_LANES), jnp.int32),
        scratch_shapes=[
            pltpu.VMEM((BB, H), jnp.float32),
            pltpu.VMEM((BB, PAD_LANES), jnp.float32),
        ],
    )
    return call(X, W_p, W_d, idx_map, zc)[:, :K]


def _knn_kernel(x_ref, p_ref, r_ref, out_ref, w_ref, vals_ref):
    b, j = pl.program_id(0), pl.program_id(1)

    @pl.when(j == 0)
    def _():
        x = x_ref[...]
        xn = x / (jnp.sqrt(jnp.sum(x * x, axis=1, keepdims=True)) + 1e-8)
        p = p_ref[...]
        pn = p / (jnp.sqrt(jnp.sum(p * p, axis=1, keepdims=True)) + 1e-8)
        sims = lax.dot_general(xn, pn, (((1,), (1,)), ((), ())),
                               preferred_element_type=jnp.float32)
        w_ref[...] = jnp.maximum(sims, 0.0)

    scores = lax.dot_general(w_ref[...], r_ref[...], (((1,), (0,)), ((), ())),
                             preferred_element_type=jnp.float32)
    bb = scores.shape[0]
    gcol = j * CHUNK + lax.broadcasted_iota(jnp.int32, (bb, CHUNK), 1)
    scores = jnp.where(gcol < N_ITEMS, scores, NEG_INF)
    n4 = CHUNK // 4
    idx_cols = [gcol[:, t * n4:(t + 1) * n4] for t in range(4)]

    first = j == 0
    prev_v = jnp.where(first, NEG_INF, vals_ref[...])
    prev_i = jnp.where(first, IDX_SENT, out_ref[...])
    vals, idx = _merge_chunk(scores, idx_cols, prev_v, prev_i)
    vals_ref[...], out_ref[...] = _pad_state(vals, idx)


def _knn_topk(X, user_ratings, user_personalities):
    call = pl.pallas_call(
        _knn_kernel,
        grid=(B // BB, N_CHUNKS),
        in_specs=[
            pl.BlockSpec((BB, D), lambda b, j: (b, 0)),
            pl.BlockSpec((N_USERS, D), lambda b, j: (0, 0)),
            pl.BlockSpec((N_USERS, CHUNK), lambda b, j: (0, j)),
        ],
        out_specs=pl.BlockSpec((BB, PAD_LANES), lambda b, j: (b, 0)),
        out_shape=jax.ShapeDtypeStruct((B, PAD_LANES), jnp.int32),
        scratch_shapes=[
            pltpu.VMEM((BB, N_USERS), jnp.float32),
            pltpu.VMEM((BB, PAD_LANES), jnp.float32),
        ],
    )
    return call(X, user_personalities, user_ratings)[:, :K]


def _smallest_unmapped(idx_map):
    present = jnp.zeros((N_ITEMS,), jnp.int32).at[idx_map].set(1)
    score = jnp.arange(N_ITEMS, dtype=jnp.int32) + present * (2 * N_ITEMS)
    neg_top, _ = lax.top_k(-score, PAD_LANES)
    return (-neg_top).reshape(1, PAD_LANES)


def kernel(X, W_sp, W_sd, W_mp, W_md, user_ratings, user_personalities,
           top_map, mid_map):
    top_map = top_map.astype(jnp.int32)
    mid_map = mid_map.astype(jnp.int32)

    n_top_pad = 2048
    n_mid_pad = 10240
    W_sd_p = jnp.pad(W_sd, ((0, 0), (0, n_top_pad - N_TOP)))
    W_md_p = jnp.pad(W_md, ((0, 0), (0, n_mid_pad - N_MID)))
    top_map_p = jnp.pad(top_map, (0, n_top_pad - N_TOP),
                        constant_values=MAP_SENT).reshape(1, n_top_pad)
    mid_map_p = jnp.pad(mid_map, (0, n_mid_pad - N_MID),
                        constant_values=MAP_SENT).reshape(1, n_mid_pad)
    zc_top = _smallest_unmapped(top_map)
    zc_mid = _smallest_unmapped(mid_map)

    top_idx = _subset_topk(X, W_sp, W_sd_p, top_map_p, zc_top, n_top_pad)
    mid_idx = _subset_topk(X, W_mp, W_md_p, mid_map_p, zc_mid, n_mid_pad)
    k_idx = _knn_topk(X, user_ratings, user_personalities)

    return jnp.concatenate(
        [top_idx[:, None, :], mid_idx[:, None, :], k_idx[:, None, :]], axis=1)


# speedup vs baseline: 2.4151x; 1.0225x over previous
import functools

import jax
import jax.numpy as jnp
from jax import lax
from jax.experimental import pallas as pl
from jax.experimental.pallas import tpu as pltpu

B = 1024
D = 32
H = 64
N_ITEMS = 100000
N_TOP = 2000
N_MID = 10000
N_USERS = 256
K = 20

BB = 256
CHUNK = 2048
QROWS = CHUNK // 128
N_CHUNKS = (N_ITEMS + CHUNK - 1) // CHUNK
PAD_LANES = 128
NG = 8
IDX_SENT = 2**31 - 1
MAP_SENT = 1 << 29
NEG_INF = float("-inf")


def _topk_extract(V, I, k):
    outs_v, outs_i = [], []
    for _ in range(k):
        m = jnp.max(V, axis=1, keepdims=True)
        tie = V == m
        ci = jnp.where(tie, I, IDX_SENT)
        si = jnp.min(ci, axis=1, keepdims=True)
        outs_v.append(m)
        outs_i.append(si)
        V = jnp.where(tie & (I == si), NEG_INF, V)
    return jnp.concatenate(outs_v, axis=1), jnp.concatenate(outs_i, axis=1)


def _ce(H, HI, i, j):
    xv, xi, yv, yi = H[i], HI[i], H[j], HI[j]
    takex = (xv > yv) | ((xv == yv) & (xi < yi))
    H[i] = jnp.where(takex, xv, yv)
    HI[i] = jnp.where(takex, xi, yi)
    H[j] = jnp.where(takex, yv, xv)
    HI[j] = jnp.where(takex, yi, xi)


def _merge_chunk(scores, idx_cols, prev_v, prev_i):
    bb, n = scores.shape
    ng = n // NG
    H = [scores[:, t * ng:(t + 1) * ng] for t in range(NG)]
    HI = list(idx_cols)
    for (i, j) in ((0, 1), (2, 3), (0, 2), (1, 3), (1, 2),
                   (4, 5), (6, 7), (4, 6), (5, 7), (5, 6),
                   (0, 4), (1, 5), (2, 6), (3, 7),
                   (2, 4), (3, 5), (1, 2), (3, 4), (5, 6)):
        _ce(H, HI, i, j)
    Bv, Bi = H[0], HI[0]
    Av, Ai = prev_v, prev_i
    outs_v, outs_i = [], []
    for _ in range(K):
        mB = jnp.max(Bv, axis=1, keepdims=True)
        mA = jnp.max(Av, axis=1, keepdims=True)
        m = jnp.maximum(mA, mB)
        tieB = Bv == m
        tieA = Av == m
        siB = jnp.min(jnp.where(tieB, Bi, IDX_SENT), axis=1, keepdims=True)
        siA = jnp.min(jnp.where(tieA, Ai, IDX_SENT), axis=1, keepdims=True)
        si = jnp.minimum(siA, siB)
        outs_v.append(m)
        outs_i.append(si)
        killA = tieA & (Ai == si)
        Av = jnp.where(killA, NEG_INF, Av)
        killB = tieB & (Bi == si)
        nv = jnp.full_like(Bv, NEG_INF)
        ni = jnp.full_like(Bi, IDX_SENT)
        for t in range(NG - 2, -1, -1):
            hit = Bi == HI[t]
            nv = jnp.where(hit, H[t + 1], nv)
            ni = jnp.where(hit, HI[t + 1], ni)
        Bv = jnp.where(killB, nv, Bv)
        Bi = jnp.where(killB, ni, Bi)
    return jnp.concatenate(outs_v, axis=1), jnp.concatenate(outs_i, axis=1)


def _pad_state(vals, idx):
    bb = vals.shape[0]
    return (jnp.concatenate(
                [vals, jnp.full((bb, PAD_LANES - K), NEG_INF, jnp.float32)],
                axis=1),
            jnp.concatenate(
                [idx, jnp.full((bb, PAD_LANES - K), IDX_SENT, jnp.int32)],
                axis=1))


def _subset_kernel(x_ref, wp_ref, wd_ref, map_ref, zc_ref, out_ref,
                   h_ref, vals_ref):
    b, j = pl.program_id(0), pl.program_id(1)

    @pl.when(j == 0)
    def _():
        h_ref[...] = jnp.tanh(
            lax.dot_general(x_ref[...], wp_ref[...], (((1,), (0,)), ((), ())),
                            preferred_element_type=jnp.float32))

    preds = lax.dot_general(h_ref[...], wd_ref[...], (((1,), (0,)), ((), ())),
                            preferred_element_type=jnp.float32)
    bb, n = preds.shape
    ng = n // NG
    idx_cols = [
        jnp.broadcast_to(map_ref[:, t * ng:(t + 1) * ng], (bb, ng))
        for t in range(NG)
    ]
    first = j == 0
    prev_v = jnp.where(first, 0.0, vals_ref[...])
    prev_i = jnp.where(first, jnp.broadcast_to(zc_ref[...], (bb, PAD_LANES)),
                       out_ref[...])
    vals, idx = _merge_chunk(preds, idx_cols, prev_v, prev_i)
    vals_ref[...], out_ref[...] = _pad_state(vals, idx)


def _subset_topk(X, W_p, W_d, idx_map, zc, n_sub_pad):
    n_chunks = n_sub_pad // CHUNK
    call = pl.pallas_call(
        _subset_kernel,
        grid=(B // BB, n_chunks),
        in_specs=[
            pl.BlockSpec((BB, D), lambda b, j: (b, 0)),
            pl.BlockSpec((D, H), lambda b, j: (0, 0)),
            pl.BlockSpec((H, CHUNK), lambda b, j: (0, j)),
            pl.BlockSpec((1, CHUNK), lambda b, j: (0, j)),
            pl.BlockSpec((1, PAD_LANES), lambda b, j: (0, 0)),
        ],
        out_specs=pl.BlockSpec((BB, PAD_LANES), lambda b, j: (b, 0)),
        out_shape=jax.ShapeDtypeStruct((B, PAD_LANES), jnp.int32),
        scratch_shapes=[
            pltpu.VMEM((BB, H), jnp.float32),
            pltpu.VMEM((BB, PAD_LANES), jnp.float32),
        ],
    )
    return call(X, W_p, W_d, idx_map, zc)[:, :K]


def _knn_kernel(x_ref, p_ref, r_ref, out_ref, w_ref, vals_ref):
    b, j = pl.program_id(0), pl.program_id(1)

    @pl.when(j == 0)
    def _():
        x = x_ref[...]
        xn = x / (jnp.sqrt(jnp.sum(x * x, axis=1, keepdims=True)) + 1e-8)
        p = p_ref[...]
        pn = p / (jnp.sqrt(jnp.sum(p * p, axis=1, keepdims=True)) + 1e-8)
        sims = lax.dot_general(xn, pn, (((1,), (1,)), ((), ())),
                               preferred_element_type=jnp.float32)
        w_ref[...] = jnp.maximum(sims, 0.0)

    scores = lax.dot_general(w_ref[...], r_ref[...], (((1,), (0,)), ((), ())),
                             preferred_element_type=jnp.float32)
    bb = scores.shape[0]
    gcol = j * CHUNK + lax.broadcasted_iota(jnp.int32, (bb, CHUNK), 1)
    scores = jnp.where(gcol < N_ITEMS, scores, NEG_INF)
    ng = CHUNK // NG
    idx_cols = [gcol[:, t * ng:(t + 1) * ng] for t in range(NG)]

    first = j == 0
    prev_v = jnp.where(first, NEG_INF, vals_ref[...])
    prev_i = jnp.where(first, IDX_SENT, out_ref[...])
    vals, idx = _merge_chunk(scores, idx_cols, prev_v, prev_i)
    vals_ref[...], out_ref[...] = _pad_state(vals, idx)


def _knn_topk(X, user_ratings, user_personalities):
    call = pl.pallas_call(
        _knn_kernel,
        grid=(B // BB, N_CHUNKS),
        in_specs=[
            pl.BlockSpec((BB, D), lambda b, j: (b, 0)),
            pl.BlockSpec((N_USERS, D), lambda b, j: (0, 0)),
            pl.BlockSpec((N_USERS, CHUNK), lambda b, j: (0, j)),
        ],
        out_specs=pl.BlockSpec((BB, PAD_LANES), lambda b, j: (b, 0)),
        out_shape=jax.ShapeDtypeStruct((B, PAD_LANES), jnp.int32),
        scratch_shapes=[
            pltpu.VMEM((BB, N_USERS), jnp.float32),
            pltpu.VMEM((BB, PAD_LANES), jnp.float32),
        ],
    )
    return call(X, user_personalities, user_ratings)[:, :K]


def _smallest_unmapped(idx_map):
    present = jnp.zeros((N_ITEMS,), jnp.int32).at[idx_map].set(1)
    score = jnp.arange(N_ITEMS, dtype=jnp.int32) + present * (2 * N_ITEMS)
    neg_top, _ = lax.top_k(-score, PAD_LANES)
    return (-neg_top).reshape(1, PAD_LANES)


def kernel(X, W_sp, W_sd, W_mp, W_md, user_ratings, user_personalities,
           top_map, mid_map):
    top_map = top_map.astype(jnp.int32)
    mid_map = mid_map.astype(jnp.int32)

    n_top_pad = 2048
    n_mid_pad = 10240
    W_sd_p = jnp.pad(W_sd, ((0, 0), (0, n_top_pad - N_TOP)))
    W_md_p = jnp.pad(W_md, ((0, 0), (0, n_mid_pad - N_MID)))
    top_map_p = jnp.pad(top_map, (0, n_top_pad - N_TOP),
                        constant_values=MAP_SENT).reshape(1, n_top_pad)
    mid_map_p = jnp.pad(mid_map, (0, n_mid_pad - N_MID),
                        constant_values=MAP_SENT).reshape(1, n_mid_pad)
    zc_top = _smallest_unmapped(top_map)
    zc_mid = _smallest_unmapped(mid_map)

    top_idx = _subset_topk(X, W_sp, W_sd_p, top_map_p, zc_top, n_top_pad)
    mid_idx = _subset_topk(X, W_mp, W_md_p, mid_map_p, zc_mid, n_mid_pad)
    k_idx = _knn_topk(X, user_ratings, user_personalities)

    return jnp.concatenate(
        [top_idx[:, None, :], mid_idx[:, None, :], k_idx[:, None, :]], axis=1)


# speedup vs baseline: 2.6809x; 1.1101x over previous
import functools

import jax
import jax.numpy as jnp
from jax import lax
from jax.experimental import pallas as pl
from jax.experimental.pallas import tpu as pltpu

B = 1024
D = 32
H = 64
N_ITEMS = 100000
N_TOP = 2000
N_MID = 10000
N_USERS = 256
K = 20

BB = 256
CHUNK = 2048
KCHUNK = 4096
KN_CHUNKS = (N_ITEMS + KCHUNK - 1) // KCHUNK
PAD_LANES = 128
NG = 8
IDX_SENT = 2**31 - 1
MAP_SENT = 1 << 29
NEG_INF = float("-inf")


def _topk_extract(V, I, k):
    outs_v, outs_i = [], []
    for _ in range(k):
        m = jnp.max(V, axis=1, keepdims=True)
        tie = V == m
        ci = jnp.where(tie, I, IDX_SENT)
        si = jnp.min(ci, axis=1, keepdims=True)
        outs_v.append(m)
        outs_i.append(si)
        V = jnp.where(tie & (I == si), NEG_INF, V)
    return jnp.concatenate(outs_v, axis=1), jnp.concatenate(outs_i, axis=1)


def _ce(H, HI, i, j):
    xv, xi, yv, yi = H[i], HI[i], H[j], HI[j]
    takex = (xv > yv) | ((xv == yv) & (xi < yi))
    H[i] = jnp.where(takex, xv, yv)
    HI[i] = jnp.where(takex, xi, yi)
    H[j] = jnp.where(takex, yv, xv)
    HI[j] = jnp.where(takex, yi, xi)


def _merge_chunk(scores, idx_cols, prev_v, prev_i):
    bb, n = scores.shape
    ng = n // NG
    H = [scores[:, t * ng:(t + 1) * ng] for t in range(NG)]
    HI = list(idx_cols)
    for (i, j) in ((0, 1), (2, 3), (0, 2), (1, 3), (1, 2),
                   (4, 5), (6, 7), (4, 6), (5, 7), (5, 6),
                   (0, 4), (1, 5), (2, 6), (3, 7),
                   (2, 4), (3, 5), (1, 2), (3, 4), (5, 6)):
        _ce(H, HI, i, j)
    Bv, Bi = H[0], HI[0]
    Av, Ai = prev_v, prev_i
    outs_v, outs_i = [], []
    for _ in range(K):
        mB = jnp.max(Bv, axis=1, keepdims=True)
        mA = jnp.max(Av, axis=1, keepdims=True)
        m = jnp.maximum(mA, mB)
        tieB = Bv == m
        tieA = Av == m
        siB = jnp.min(jnp.where(tieB, Bi, IDX_SENT), axis=1, keepdims=True)
        siA = jnp.min(jnp.where(tieA, Ai, IDX_SENT), axis=1, keepdims=True)
        si = jnp.minimum(siA, siB)
        outs_v.append(m)
        outs_i.append(si)
        killA = tieA & (Ai == si)
        Av = jnp.where(killA, NEG_INF, Av)
        killB = tieB & (Bi == si)
        nv = jnp.full_like(Bv, NEG_INF)
        ni = jnp.full_like(Bi, IDX_SENT)
        for t in range(NG - 2, -1, -1):
            hit = Bi == HI[t]
            nv = jnp.where(hit, H[t + 1], nv)
            ni = jnp.where(hit, HI[t + 1], ni)
        Bv = jnp.where(killB, nv, Bv)
        Bi = jnp.where(killB, ni, Bi)
    return jnp.concatenate(outs_v, axis=1), jnp.concatenate(outs_i, axis=1)


def _pad_state(vals, idx):
    bb = vals.shape[0]
    return (jnp.concatenate(
                [vals, jnp.full((bb, PAD_LANES - K), NEG_INF, jnp.float32)],
                axis=1),
            jnp.concatenate(
                [idx, jnp.full((bb, PAD_LANES - K), IDX_SENT, jnp.int32)],
                axis=1))


def _subset_kernel(x_ref, wp_ref, wd_ref, map_ref, zc_ref, out_ref,
                   h_ref, vals_ref):
    b, j = pl.program_id(0), pl.program_id(1)

    @pl.when(j == 0)
    def _():
        h_ref[...] = jnp.tanh(
            lax.dot_general(x_ref[...], wp_ref[...], (((1,), (0,)), ((), ())),
                            preferred_element_type=jnp.float32))

    preds = lax.dot_general(h_ref[...], wd_ref[...], (((1,), (0,)), ((), ())),
                            preferred_element_type=jnp.float32)
    bb, n = preds.shape
    ng = n // NG
    idx_cols = [
        jnp.broadcast_to(map_ref[:, t * ng:(t + 1) * ng], (bb, ng))
        for t in range(NG)
    ]
    first = j == 0
    prev_v = jnp.where(first, 0.0, vals_ref[...])
    prev_i = jnp.where(first, jnp.broadcast_to(zc_ref[...], (bb, PAD_LANES)),
                       out_ref[...])
    vals, idx = _merge_chunk(preds, idx_cols, prev_v, prev_i)
    vals_ref[...], out_ref[...] = _pad_state(vals, idx)


def _subset_topk(X, W_p, W_d, idx_map, zc, n_sub_pad):
    n_chunks = n_sub_pad // CHUNK
    call = pl.pallas_call(
        _subset_kernel,
        grid=(B // BB, n_chunks),
        in_specs=[
            pl.BlockSpec((BB, D), lambda b, j: (b, 0)),
            pl.BlockSpec((D, H), lambda b, j: (0, 0)),
            pl.BlockSpec((H, CHUNK), lambda b, j: (0, j)),
            pl.BlockSpec((1, CHUNK), lambda b, j: (0, j)),
            pl.BlockSpec((1, PAD_LANES), lambda b, j: (0, 0)),
        ],
        out_specs=pl.BlockSpec((BB, PAD_LANES), lambda b, j: (b, 0)),
        out_shape=jax.ShapeDtypeStruct((B, PAD_LANES), jnp.int32),
        scratch_shapes=[
            pltpu.VMEM((BB, H), jnp.float32),
            pltpu.VMEM((BB, PAD_LANES), jnp.float32),
        ],
    )
    return call(X, W_p, W_d, idx_map, zc)[:, :K]


def _knn_kernel(x_ref, p_ref, r_ref, out_ref, w_ref, vals_ref):
    b, j = pl.program_id(0), pl.program_id(1)

    @pl.when(j == 0)
    def _():
        x = x_ref[...]
        xn = x / (jnp.sqrt(jnp.sum(x * x, axis=1, keepdims=True)) + 1e-8)
        p = p_ref[...]
        pn = p / (jnp.sqrt(jnp.sum(p * p, axis=1, keepdims=True)) + 1e-8)
        sims = lax.dot_general(xn, pn, (((1,), (1,)), ((), ())),
                               preferred_element_type=jnp.float32)
        w_ref[...] = jnp.maximum(sims, 0.0)

    scores = lax.dot_general(w_ref[...], r_ref[...], (((1,), (0,)), ((), ())),
                             preferred_element_type=jnp.float32)
    bb = scores.shape[0]
    gcol = j * KCHUNK + lax.broadcasted_iota(jnp.int32, (bb, KCHUNK), 1)
    scores = jnp.where(gcol < N_ITEMS, scores, NEG_INF)
    ng = KCHUNK // NG
    idx_cols = [gcol[:, t * ng:(t + 1) * ng] for t in range(NG)]

    first = j == 0
    prev_v = jnp.where(first, NEG_INF, vals_ref[...])
    prev_i = jnp.where(first, IDX_SENT, out_ref[...])
    vals, idx = _merge_chunk(scores, idx_cols, prev_v, prev_i)
    vals_ref[...], out_ref[...] = _pad_state(vals, idx)


def _knn_topk(X, user_ratings, user_personalities):
    call = pl.pallas_call(
        _knn_kernel,
        grid=(B // BB, KN_CHUNKS),
        in_specs=[
            pl.BlockSpec((BB, D), lambda b, j: (b, 0)),
            pl.BlockSpec((N_USERS, D), lambda b, j: (0, 0)),
            pl.BlockSpec((N_USERS, KCHUNK), lambda b, j: (0, j)),
        ],
        out_specs=pl.BlockSpec((BB, PAD_LANES), lambda b, j: (b, 0)),
        out_shape=jax.ShapeDtypeStruct((B, PAD_LANES), jnp.int32),
        scratch_shapes=[
            pltpu.VMEM((BB, N_USERS), jnp.float32),
            pltpu.VMEM((BB, PAD_LANES), jnp.float32),
        ],
    )
    return call(X, user_personalities, user_ratings)[:, :K]


def _smallest_unmapped(idx_map):
    present = jnp.zeros((N_ITEMS,), jnp.int32).at[idx_map].set(1)
    score = jnp.arange(N_ITEMS, dtype=jnp.int32) + present * (2 * N_ITEMS)
    neg_top, _ = lax.top_k(-score, PAD_LANES)
    return (-neg_top).reshape(1, PAD_LANES)


def kernel(X, W_sp, W_sd, W_mp, W_md, user_ratings, user_personalities,
           top_map, mid_map):
    top_map = top_map.astype(jnp.int32)
    mid_map = mid_map.astype(jnp.int32)

    n_top_pad = 2048
    n_mid_pad = 10240
    W_sd_p = jnp.pad(W_sd, ((0, 0), (0, n_top_pad - N_TOP)))
    W_md_p = jnp.pad(W_md, ((0, 0), (0, n_mid_pad - N_MID)))
    top_map_p = jnp.pad(top_map, (0, n_top_pad - N_TOP),
                        constant_values=MAP_SENT).reshape(1, n_top_pad)
    mid_map_p = jnp.pad(mid_map, (0, n_mid_pad - N_MID),
                        constant_values=MAP_SENT).reshape(1, n_mid_pad)
    zc_top = _smallest_unmapped(top_map)
    zc_mid = _smallest_unmapped(mid_map)

    top_idx = _subset_topk(X, W_sp, W_sd_p, top_map_p, zc_top, n_top_pad)
    mid_idx = _subset_topk(X, W_mp, W_md_p, mid_map_p, zc_mid, n_mid_pad)
    k_idx = _knn_topk(X, user_ratings, user_personalities)

    return jnp.concatenate(
        [top_idx[:, None, :], mid_idx[:, None, :], k_idx[:, None, :]], axis=1)
